# baseline (speedup 1.0000x reference)
import jax
import jax.numpy as jnp
from jax import lax
from jax.experimental import pallas as pl
from jax.experimental.pallas import tpu as pltpu

WIDTH = 1000
FEATURE_DIM = 100000
ROW_BLOCK = 16
NBUF = 4


def _body(state_ref, out_ref, b0, b1, b2, b3, s0, s1, s2, s3):
    bufs = (b0, b1, b2, b3)
    sems = (s0, s1, s2, s3)
    n = state_ref.shape[0]
    nsteps = n // ROW_BLOCK
    cols = jax.lax.broadcasted_iota(jnp.int32, (ROW_BLOCK, FEATURE_DIM), 1)

    def fill(step, b, wait_first):
        r0 = step * ROW_BLOCK
        if wait_first:
            pltpu.make_async_copy(
                bufs[b], out_ref.at[pl.ds((step - NBUF) * ROW_BLOCK, ROW_BLOCK), :],
                sems[b]).wait()
        idx = state_ref[pl.ds(r0, ROW_BLOCK), 0] + WIDTH * state_ref[pl.ds(r0, ROW_BLOCK), 1]
        bufs[b][...] = (cols == idx[:, None]).astype(jnp.float32)
        pltpu.make_async_copy(
            bufs[b], out_ref.at[pl.ds(r0, ROW_BLOCK), :], sems[b]).start()

    for b in range(NBUF):
        fill(b, b, False)

    def loop(g, carry):
        for b in range(NBUF):
            fill(g * NBUF + b, b, True)
        return carry

    lax.fori_loop(1, nsteps // NBUF, loop, 0)

    for b in range(NBUF):
        step = nsteps - NBUF + b
        pltpu.make_async_copy(
            bufs[b], out_ref.at[pl.ds(step * ROW_BLOCK, ROW_BLOCK), :],
            sems[b]).wait()


def kernel(state):
    n = state.shape[0]
    return pl.pallas_call(
        _body,
        in_specs=[pl.BlockSpec(memory_space=pltpu.VMEM)],
        out_specs=pl.BlockSpec(memory_space=pltpu.MemorySpace.HBM),
        out_shape=jax.ShapeDtypeStruct((n, FEATURE_DIM), jnp.float32),
        scratch_shapes=(
            [pltpu.VMEM((ROW_BLOCK, FEATURE_DIM), jnp.float32)] * NBUF
            + [pltpu.SemaphoreType.DMA] * NBUF
        ),
    )(state)


# speedup vs baseline: 1.0052x; 1.0052x over previous
import jax
import jax.numpy as jnp
from jax.experimental import pallas as pl

WIDTH = 1000
FEATURE_DIM = 100000
ROW_BLOCK = 16


def _onehot_block(state_ref, out_ref):
    idx = state_ref[:, 0] + WIDTH * state_ref[:, 1]
    cols = jax.lax.broadcasted_iota(jnp.int32, out_ref.shape, 1)
    out_ref[...] = (cols == idx[:, None]).astype(jnp.float32)


def kernel(state):
    n = state.shape[0]
    grid = n // ROW_BLOCK
    return pl.pallas_call(
        _onehot_block,
        grid=(grid,),
        in_specs=[pl.BlockSpec((ROW_BLOCK, 2), lambda i: (i, 0))],
        out_specs=pl.BlockSpec((ROW_BLOCK, FEATURE_DIM), lambda i: (i, 0)),
        out_shape=jax.ShapeDtypeStruct((n, FEATURE_DIM), jnp.float32),
    )(state)
